# Initial kernel scaffold; baseline (speedup 1.0000x reference)
import jax
import jax.numpy as jnp
from jax.experimental import pallas as pl


def kernel(x, edge_index, W_self1, W_neigh1, b1, W_self2, W_neigh2, b2):
    raise NotImplementedError("write your pallas kernel here")


# speedup vs baseline: 11.4288x; 11.4288x over previous
import functools

import jax
import jax.numpy as jnp
from jax import lax
from jax.experimental import pallas as pl
from jax.experimental.pallas import tpu as pltpu
from jax.experimental.pallas import tpu_sc as plsc

NC = 2
NS = 16
NW = NC * NS
K = 128
G = 8
ZR = 64


def _make_agg(n_acc, e_pad, d, with_deg):
  n_chunks = e_pad // (NW * K)
  rpt = n_acc // NS
  mesh = plsc.VectorSubcoreMesh(core_axis_name="c", subcore_axis_name="s",
                                num_cores=NC, num_subcores=NS)
  out_type = [jax.ShapeDtypeStruct((n_acc, d), jnp.float32),
              jax.ShapeDtypeStruct((n_acc, d), jnp.float32)]
  if with_deg:
    out_type += [jax.ShapeDtypeStruct((n_acc,), jnp.float32),
                 jax.ShapeDtypeStruct((n_acc,), jnp.float32)]
  scratch = [pltpu.VMEM((G, K), jnp.int32),
             pltpu.VMEM((G, K), jnp.int32),
             pltpu.VMEM((2, K, d), jnp.float32),
             pltpu.VMEM((ZR, d), jnp.float32),
             pltpu.VMEM_SHARED((n_acc, d), jnp.float32),
             pltpu.SemaphoreType.DMA,
             pltpu.SemaphoreType.DMA]
  if with_deg:
    scratch += [pltpu.VMEM((K,), jnp.float32),
                pltpu.VMEM((rpt,), jnp.float32),
                pltpu.VMEM_SHARED((n_acc,), jnp.float32)]

  @functools.partial(pl.kernel, mesh=mesh, out_type=out_type,
                     scratch_types=scratch,
                     compiler_params=pltpu.CompilerParams(
                         use_tc_tiling_on_sc=False))
  def agg(p_hbm, src_hbm, dst_hbm, *refs):
    if with_deg:
      (out_a, out_b, deg_a, deg_b, src_idx, dst_idx, rows, zblk, acc, sem0,
       sem1, ones, dzero, dacc) = refs
    else:
      (out_a, out_b, src_idx, dst_idx, rows, zblk, acc, sem0, sem1) = refs
    sems = (sem0, sem1)
    c = lax.axis_index("c")
    s = lax.axis_index("s")
    wid = c * NS + s
    row0 = s * rpt

    def zrow(i, _):
      for kk in range(d // 16):
        zblk[i, pl.ds(kk * 16, 16)] = jnp.zeros((16,), jnp.float32)
      return 0
    lax.fori_loop(0, ZR, zrow, 0)

    def zacc(i, _):
      pltpu.sync_copy(zblk, acc.at[pl.ds(row0 + i * ZR, ZR)])
      return 0
    lax.fori_loop(0, rpt // ZR, zacc, 0)

    if with_deg:
      for kk in range(K // 16):
        ones[pl.ds(kk * 16, 16)] = jnp.ones((16,), jnp.float32)

      def zdeg(i, _):
        dzero[pl.ds(i * 16, 16)] = jnp.zeros((16,), jnp.float32)
        return 0
      lax.fori_loop(0, rpt // 16, zdeg, 0)
      pltpu.sync_copy(dzero, dacc.at[pl.ds(row0, rpt)])
    plsc.subcore_barrier()

    crow = wid * n_chunks

    def group(g, _):
      r0 = crow + g * G
      pltpu.sync_copy(src_hbm.at[pl.ds(r0, G)], src_idx)
      pltpu.sync_copy(dst_hbm.at[pl.ds(r0, G)], dst_idx)
      desc = pltpu.async_copy(p_hbm.at[src_idx.at[0]], rows.at[0], sems[0])
      for k in range(G):
        b = k & 1
        nxt = None
        if k + 1 < G:
          nxt = pltpu.async_copy(p_hbm.at[src_idx.at[k + 1]],
                                 rows.at[1 - b], sems[1 - b])
        desc.wait()
        pltpu.sync_copy(rows.at[b], acc.at[dst_idx.at[k]], add=True)
        if with_deg:
          pltpu.sync_copy(ones, dacc.at[dst_idx.at[k]], add=True)
        desc = nxt
      return 0
    lax.fori_loop(0, n_chunks // G, group, 0)
    plsc.subcore_barrier()

    @pl.when(c == 0)
    def _():
      pltpu.sync_copy(acc.at[pl.ds(row0, rpt)], out_a.at[pl.ds(row0, rpt)])
      if with_deg:
        pltpu.sync_copy(dacc.at[pl.ds(row0, rpt)], deg_a.at[pl.ds(row0, rpt)])

    @pl.when(c == 1)
    def _():
      pltpu.sync_copy(acc.at[pl.ds(row0, rpt)], out_b.at[pl.ds(row0, rpt)])
      if with_deg:
        pltpu.sync_copy(dacc.at[pl.ds(row0, rpt)], deg_b.at[pl.ds(row0, rpt)])

  return agg


def _mm(x, w, block):
  n, din = x.shape
  dout = w.shape[1]

  def body(x_ref, w_ref, o_ref):
    o_ref[...] = jnp.dot(x_ref[...], w_ref[...],
                         preferred_element_type=jnp.float32)

  return pl.pallas_call(
      body,
      grid=(n // block,),
      in_specs=[pl.BlockSpec((block, din), lambda i: (i, 0)),
                pl.BlockSpec((din, dout), lambda i: (0, 0))],
      out_specs=pl.BlockSpec((block, dout), lambda i: (i, 0)),
      out_shape=jax.ShapeDtypeStruct((n, dout), jnp.float32),
  )(x, w)


def _h1_p2(x, w_self1, b1, a1a, a1b, dga, dgb, w_neigh2, block):
  n, din = x.shape
  dh = w_self1.shape[1]
  d2 = w_neigh2.shape[1]
  n_acc = a1a.shape[0]

  def body(x_ref, ws_ref, b_ref, aa_ref, ab_ref, da_ref, db_ref, wn_ref,
           h_ref, p2_ref):
    inv = 1.0 / jnp.maximum(da_ref[...] + db_ref[...], 1.0)
    h = (jnp.dot(x_ref[...], ws_ref[...], preferred_element_type=jnp.float32)
         + b_ref[...] + (aa_ref[...] + ab_ref[...]) * inv)
    h = jnp.maximum(h, 0.0)
    h_ref[...] = h
    p2_ref[...] = jnp.dot(h, wn_ref[...], preferred_element_type=jnp.float32)

  return pl.pallas_call(
      body,
      grid=(n // block,),
      in_specs=[pl.BlockSpec((block, din), lambda i: (i, 0)),
                pl.BlockSpec((din, dh), lambda i: (0, 0)),
                pl.BlockSpec((1, dh), lambda i: (0, 0)),
                pl.BlockSpec((block, dh), lambda i: (i, 0)),
                pl.BlockSpec((block, dh), lambda i: (i, 0)),
                pl.BlockSpec((block, 1), lambda i: (i, 0)),
                pl.BlockSpec((block, 1), lambda i: (i, 0)),
                pl.BlockSpec((dh, d2), lambda i: (0, 0))],
      out_specs=[pl.BlockSpec((block, dh), lambda i: (i, 0)),
                 pl.BlockSpec((block, d2), lambda i: (i, 0))],
      out_shape=[jax.ShapeDtypeStruct((n, dh), jnp.float32),
                 jax.ShapeDtypeStruct((n, d2), jnp.float32)],
  )(x, w_self1, b1, a1a, a1b, dga, dgb, w_neigh2)


def _out_layer(h1, w_self2, b2, a2a, a2b, dga, dgb, block):
  n, dh = h1.shape
  d2 = w_self2.shape[1]

  def body(h_ref, ws_ref, b_ref, aa_ref, ab_ref, da_ref, db_ref, o_ref):
    inv = 1.0 / jnp.maximum(da_ref[...] + db_ref[...], 1.0)
    o_ref[...] = (jnp.dot(h_ref[...], ws_ref[...],
                          preferred_element_type=jnp.float32)
                  + b_ref[...] + (aa_ref[...] + ab_ref[...]) * inv)

  return pl.pallas_call(
      body,
      grid=(n // block,),
      in_specs=[pl.BlockSpec((block, dh), lambda i: (i, 0)),
                pl.BlockSpec((dh, d2), lambda i: (0, 0)),
                pl.BlockSpec((1, d2), lambda i: (0, 0)),
                pl.BlockSpec((block, d2), lambda i: (i, 0)),
                pl.BlockSpec((block, d2), lambda i: (i, 0)),
                pl.BlockSpec((block, 1), lambda i: (i, 0)),
                pl.BlockSpec((block, 1), lambda i: (i, 0))],
      out_specs=pl.BlockSpec((block, d2), lambda i: (i, 0)),
      out_shape=jax.ShapeDtypeStruct((n, d2), jnp.float32),
  )(h1, w_self2, b2, a2a, a2b, dga, dgb)


def kernel(x, edge_index, W_self1, W_neigh1, b1, W_self2, W_neigh2, b2):
  n, _ = x.shape
  e = edge_index.shape[1]
  dh = W_neigh1.shape[1]
  dout = W_neigh2.shape[1]

  rpt = -(-(n + 1) // (NS * ZR)) * ZR
  n_acc = NS * rpt
  e_pad = -(-e // (NW * K * G)) * (NW * K * G)

  src = edge_index[0].astype(jnp.int32)
  dst = edge_index[1].astype(jnp.int32)
  pad = e_pad - e
  if pad:
    fill = jnp.arange(pad, dtype=jnp.int32)
    src = jnp.concatenate([src, fill % n])
    dst = jnp.concatenate([dst, n + fill % (n_acc - n)])
  src = src.reshape(e_pad // K, K)
  dst = dst.reshape(e_pad // K, K)

  block = 400 if n % 400 == 0 else 8

  p1 = _mm(x, W_neigh1, block)
  a1a, a1b, dga, dgb = _make_agg(n_acc, e_pad, dh, True)(p1, src, dst)
  dga = dga.reshape(n_acc, 1)
  dgb = dgb.reshape(n_acc, 1)
  h1, p2 = _h1_p2(x, W_self1, b1.reshape(1, dh), a1a, a1b, dga, dgb,
                  W_neigh2, block)

  a2a, a2b = _make_agg(n_acc, e_pad, dout, False)(p2, src, dst)
  return _out_layer(h1, W_self2, b2.reshape(1, dout), a2a, a2b, dga, dgb,
                    block)


# speedup vs baseline: 12.5737x; 1.1002x over previous
import functools

import jax
import jax.numpy as jnp
from jax import lax
from jax.experimental import pallas as pl
from jax.experimental.pallas import tpu as pltpu
from jax.experimental.pallas import tpu_sc as plsc

NC = 2
NS = 16
NW = NC * NS
K = 128
G = 8
ZR = 64


def _make_agg(n_acc, e_pad, d, with_deg, tc_tiling):
  n_chunks = e_pad // (NW * K)
  rpt = n_acc // NS
  mesh = plsc.VectorSubcoreMesh(core_axis_name="c", subcore_axis_name="s",
                                num_cores=NC, num_subcores=NS)
  out_type = [jax.ShapeDtypeStruct((n_acc, d), jnp.float32),
              jax.ShapeDtypeStruct((n_acc, d), jnp.float32)]
  if with_deg:
    out_type += [jax.ShapeDtypeStruct((n_acc,), jnp.float32),
                 jax.ShapeDtypeStruct((n_acc,), jnp.float32)]
  scratch = [pltpu.VMEM((G, K), jnp.int32),
             pltpu.VMEM((G, K), jnp.int32),
             pltpu.VMEM((2, K, d), jnp.float32),
             pltpu.VMEM((ZR, d), jnp.float32),
             pltpu.VMEM_SHARED((n_acc, d), jnp.float32),
             pltpu.SemaphoreType.DMA,
             pltpu.SemaphoreType.DMA]
  if with_deg:
    scratch += [pltpu.VMEM((K,), jnp.float32),
                pltpu.VMEM((rpt,), jnp.float32),
                pltpu.VMEM_SHARED((n_acc,), jnp.float32)]

  @functools.partial(pl.kernel, mesh=mesh, out_type=out_type,
                     scratch_types=scratch,
                     compiler_params=pltpu.CompilerParams(
                         use_tc_tiling_on_sc=tc_tiling))
  def agg(p_hbm, src_hbm, dst_hbm, *refs):
    if with_deg:
      (out_a, out_b, deg_a, deg_b, src_idx, dst_idx, rows, zblk, acc, sem0,
       sem1, ones, dzero, dacc) = refs
    else:
      (out_a, out_b, src_idx, dst_idx, rows, zblk, acc, sem0, sem1) = refs
    sems = (sem0, sem1)
    c = lax.axis_index("c")
    s = lax.axis_index("s")
    wid = c * NS + s
    row0 = s * rpt

    def zrow(i, _):
      for kk in range(d // 16):
        zblk[i, pl.ds(kk * 16, 16)] = jnp.zeros((16,), jnp.float32)
      return 0
    lax.fori_loop(0, ZR, zrow, 0)

    def zacc(i, _):
      pltpu.sync_copy(zblk, acc.at[pl.ds(row0 + i * ZR, ZR)])
      return 0
    lax.fori_loop(0, rpt // ZR, zacc, 0)

    if with_deg:
      for kk in range(K // 16):
        ones[pl.ds(kk * 16, 16)] = jnp.ones((16,), jnp.float32)

      def zdeg(i, _):
        dzero[pl.ds(i * 16, 16)] = jnp.zeros((16,), jnp.float32)
        return 0
      lax.fori_loop(0, rpt // 16, zdeg, 0)
      pltpu.sync_copy(dzero, dacc.at[pl.ds(row0, rpt)])
    plsc.subcore_barrier()

    crow = wid * n_chunks

    def group(g, _):
      r0 = crow + g * G
      pltpu.sync_copy(src_hbm.at[pl.ds(r0, G)], src_idx)
      pltpu.sync_copy(dst_hbm.at[pl.ds(r0, G)], dst_idx)
      desc = pltpu.async_copy(p_hbm.at[src_idx.at[0]], rows.at[0], sems[0])
      for k in range(G):
        b = k & 1
        nxt = None
        if k + 1 < G:
          nxt = pltpu.async_copy(p_hbm.at[src_idx.at[k + 1]],
                                 rows.at[1 - b], sems[1 - b])
        desc.wait()
        pltpu.sync_copy(rows.at[b], acc.at[dst_idx.at[k]], add=True)
        if with_deg:
          pltpu.sync_copy(ones, dacc.at[dst_idx.at[k]], add=True)
        desc = nxt
      return 0
    lax.fori_loop(0, n_chunks // G, group, 0)
    plsc.subcore_barrier()

    @pl.when(c == 0)
    def _():
      pltpu.sync_copy(acc.at[pl.ds(row0, rpt)], out_a.at[pl.ds(row0, rpt)])
      if with_deg:
        pltpu.sync_copy(dacc.at[pl.ds(row0, rpt)], deg_a.at[pl.ds(row0, rpt)])

    @pl.when(c == 1)
    def _():
      pltpu.sync_copy(acc.at[pl.ds(row0, rpt)], out_b.at[pl.ds(row0, rpt)])
      if with_deg:
        pltpu.sync_copy(dacc.at[pl.ds(row0, rpt)], deg_b.at[pl.ds(row0, rpt)])

  return agg


def _mm(x, w, block):
  n, din = x.shape
  dout = w.shape[1]

  def body(x_ref, w_ref, o_ref):
    o_ref[...] = jnp.dot(x_ref[...], w_ref[...],
                         preferred_element_type=jnp.float32)

  return pl.pallas_call(
      body,
      grid=(n // block,),
      in_specs=[pl.BlockSpec((block, din), lambda i: (i, 0)),
                pl.BlockSpec((din, dout), lambda i: (0, 0))],
      out_specs=pl.BlockSpec((block, dout), lambda i: (i, 0)),
      out_shape=jax.ShapeDtypeStruct((n, dout), jnp.float32),
  )(x, w)


def _mm_bias(x, w, b, block):
  n, din = x.shape
  dout = w.shape[1]

  def body(x_ref, w_ref, b_ref, o_ref):
    o_ref[...] = jnp.dot(x_ref[...], w_ref[...],
                         preferred_element_type=jnp.float32) + b_ref[...]

  return pl.pallas_call(
      body,
      grid=(n // block,),
      in_specs=[pl.BlockSpec((block, din), lambda i: (i, 0)),
                pl.BlockSpec((din, dout), lambda i: (0, 0)),
                pl.BlockSpec((1, dout), lambda i: (0, 0))],
      out_specs=pl.BlockSpec((block, dout), lambda i: (i, 0)),
      out_shape=jax.ShapeDtypeStruct((n, dout), jnp.float32),
  )(x, w, b)


def _combine1(self1, a1a, a1b, dga, dgb, w_neigh2, block):
  n, dh = self1.shape
  d2 = w_neigh2.shape[1]

  def body(s_ref, aa_ref, ab_ref, da_ref, db_ref, wn_ref, h_ref, p2_ref):
    inv = 1.0 / jnp.maximum(da_ref[...] + db_ref[...], 1.0)
    h = jnp.maximum(s_ref[...] + (aa_ref[...] + ab_ref[...]) * inv, 0.0)
    h_ref[...] = h
    p2_ref[...] = jnp.dot(h, wn_ref[...], preferred_element_type=jnp.float32)

  return pl.pallas_call(
      body,
      grid=(n // block,),
      in_specs=[pl.BlockSpec((block, dh), lambda i: (i, 0)),
                pl.BlockSpec((block, dh), lambda i: (i, 0)),
                pl.BlockSpec((block, dh), lambda i: (i, 0)),
                pl.BlockSpec((block, 1), lambda i: (i, 0)),
                pl.BlockSpec((block, 1), lambda i: (i, 0)),
                pl.BlockSpec((dh, d2), lambda i: (0, 0))],
      out_specs=[pl.BlockSpec((block, dh), lambda i: (i, 0)),
                 pl.BlockSpec((block, d2), lambda i: (i, 0))],
      out_shape=[jax.ShapeDtypeStruct((n, dh), jnp.float32),
                 jax.ShapeDtypeStruct((n, d2), jnp.float32)],
  )(self1, a1a, a1b, dga, dgb, w_neigh2)


def _combine2(self2, a2a, a2b, dga, dgb, block):
  n, d2 = self2.shape

  def body(s_ref, aa_ref, ab_ref, da_ref, db_ref, o_ref):
    inv = 1.0 / jnp.maximum(da_ref[...] + db_ref[...], 1.0)
    o_ref[...] = s_ref[...] + (aa_ref[...] + ab_ref[...]) * inv

  return pl.pallas_call(
      body,
      grid=(n // block,),
      in_specs=[pl.BlockSpec((block, d2), lambda i: (i, 0)),
                pl.BlockSpec((block, d2), lambda i: (i, 0)),
                pl.BlockSpec((block, d2), lambda i: (i, 0)),
                pl.BlockSpec((block, 1), lambda i: (i, 0)),
                pl.BlockSpec((block, 1), lambda i: (i, 0))],
      out_specs=pl.BlockSpec((block, d2), lambda i: (i, 0)),
      out_shape=jax.ShapeDtypeStruct((n, d2), jnp.float32),
  )(self2, a2a, a2b, dga, dgb)


def kernel(x, edge_index, W_self1, W_neigh1, b1, W_self2, W_neigh2, b2):
  n, _ = x.shape
  e = edge_index.shape[1]
  dh = W_neigh1.shape[1]
  dout = W_neigh2.shape[1]

  rpt = -(-(n + 1) // (NS * ZR)) * ZR
  n_acc = NS * rpt
  e_pad = -(-e // (NW * K * G)) * (NW * K * G)

  src = edge_index[0].astype(jnp.int32)
  dst = edge_index[1].astype(jnp.int32)
  pad = e_pad - e
  if pad:
    fill = jnp.arange(pad, dtype=jnp.int32)
    src = jnp.concatenate([src, (fill & 8191) if n > 8191 else (fill % n)])
    dst = jnp.concatenate([dst, n + jnp.minimum(fill & 127, n_acc - n - 1)])
  src = src.reshape(e_pad // K, K)
  dst = dst.reshape(e_pad // K, K)

  block = 2000 if n % 2000 == 0 else 8

  p1 = _mm(x, W_neigh1, block)
  a1a, a1b, dga, dgb = _make_agg(n_acc, e_pad, dh, True, True)(p1, src, dst)
  self1 = _mm_bias(x, W_self1, b1.reshape(1, dh), block)
  dga = dga.reshape(n_acc, 1)
  dgb = dgb.reshape(n_acc, 1)
  h1, p2 = _combine1(self1, a1a, a1b, dga, dgb, W_neigh2, block)

  a2a, a2b = _make_agg(n_acc, e_pad, dout, False, False)(p2, src, dst)
  self2 = _mm_bias(h1, W_self2, b2.reshape(1, dout), block)
  return _combine2(self2, a2a, a2b, dga, dgb, block)


# speedup vs baseline: 13.5463x; 1.0774x over previous
import functools

import jax
import jax.numpy as jnp
from jax import lax
from jax.experimental import pallas as pl
from jax.experimental.pallas import tpu as pltpu
from jax.experimental.pallas import tpu_sc as plsc

NC = 2
NS = 16
NW = NC * NS
K = 128
G = 16
NB = 6
GA = 3
ZR = 64


def _make_agg(n_acc, e_pad, d2, with_deg):
  n_chunks = e_pad // (NS * K)
  rpt = n_acc // NS
  mesh = plsc.VectorSubcoreMesh(core_axis_name="c", subcore_axis_name="s",
                                num_cores=NC, num_subcores=NS)
  out_type = [jax.ShapeDtypeStruct((n_acc, 2 * d2), jnp.float32)]
  if with_deg:
    out_type += [jax.ShapeDtypeStruct((n_acc,), jnp.float32)]
  scratch = [pltpu.VMEM((G, K), jnp.int32),
             pltpu.VMEM((G, K), jnp.int32),
             pltpu.VMEM((NB, K, d2), jnp.float32),
             pltpu.VMEM((ZR, d2), jnp.float32),
             pltpu.VMEM_SHARED((n_acc, d2), jnp.float32)]
  scratch += [pltpu.SemaphoreType.DMA] * (2 * NB + 1)
  if with_deg:
    scratch += [pltpu.VMEM((K,), jnp.float32),
                pltpu.VMEM((rpt,), jnp.float32),
                pltpu.VMEM_SHARED((n_acc,), jnp.float32)]

  @functools.partial(pl.kernel, mesh=mesh, out_type=out_type,
                     scratch_types=scratch,
                     compiler_params=pltpu.CompilerParams(
                         use_tc_tiling_on_sc=False))
  def agg(p_lo, p_hi, src_hbm, dst_hbm, *refs):
    nsem = 2 * NB + 1
    if with_deg:
      (out, deg_o, src_idx, dst_idx, rows, zblk, acc) = refs[:7]
      sems = refs[7:7 + nsem]
      ones, dzero, dacc = refs[7 + nsem:]
    else:
      (out, src_idx, dst_idx, rows, zblk, acc) = refs[:6]
      sems = refs[6:6 + nsem]
    gsems, ssems, dsem = sems[:NB], sems[NB:2 * NB], sems[2 * NB]
    c = lax.axis_index("c")
    s = lax.axis_index("s")
    row0 = s * rpt

    def zrow(i, _):
      for kk in range(d2 // 16):
        zblk[i, pl.ds(kk * 16, 16)] = jnp.zeros((16,), jnp.float32)
      return 0
    lax.fori_loop(0, ZR, zrow, 0)

    def zacc(i, _):
      pltpu.sync_copy(zblk, acc.at[pl.ds(row0 + i * ZR, ZR)])
      return 0
    lax.fori_loop(0, rpt // ZR, zacc, 0)

    if with_deg:
      for kk in range(K // 16):
        ones[pl.ds(kk * 16, 16)] = jnp.ones((16,), jnp.float32)

      def zdeg(i, _):
        dzero[pl.ds(i * 16, 16)] = jnp.zeros((16,), jnp.float32)
        return 0
      lax.fori_loop(0, rpt // 16, zdeg, 0)
      pltpu.sync_copy(dzero, dacc.at[pl.ds(row0, rpt)])
    plsc.subcore_barrier()

    crow = s * n_chunks

    def group(g, _):
      r0 = crow + g * G
      pltpu.sync_copy(src_hbm.at[pl.ds(r0, G)], src_idx)
      pltpu.sync_copy(dst_hbm.at[pl.ds(r0, G)], dst_idx)
      sd, dd = {}, {}
      waited = set()
      state = {"nf": 0}

      def fire_until(limit):
        while state["nf"] < G and state["nf"] < limit:
          j = state["nf"]
          prev = j - NB
          if prev >= 0 and prev not in waited:
            sd[prev].wait()
            waited.add(prev)
          idx_r = src_idx.at[j]
          buf_r = rows.at[j % NB]
          sem = gsems[j % NB]

          @pl.when(c == 0)
          def _():
            pltpu.async_copy(p_lo.at[idx_r], buf_r, sem)

          @pl.when(c == 1)
          def _():
            pltpu.async_copy(p_hi.at[idx_r], buf_r, sem)
          state["nf"] = j + 1

      fire_until(GA)
      for k in range(G):
        fire_until(k + 1 + GA)
        pltpu.make_async_copy(p_lo.at[src_idx.at[k]], rows.at[k % NB],
                              gsems[k % NB]).wait()
        sd[k] = pltpu.async_copy(rows.at[k % NB], acc.at[dst_idx.at[k]],
                                 ssems[k % NB], add=True)
        if with_deg:
          dd[k] = pltpu.async_copy(ones, dacc.at[dst_idx.at[k]], dsem,
                                   add=True)
      for k in range(G):
        if k not in waited:
          sd[k].wait()
        if with_deg:
          dd[k].wait()
      return 0
    lax.fori_loop(0, n_chunks // G, group, 0)
    plsc.subcore_barrier()

    d2c = c * d2
    pltpu.sync_copy(acc.at[pl.ds(row0, rpt)],
                    out.at[pl.ds(row0, rpt), pl.ds(d2c, d2)])
    if with_deg:
      @pl.when(c == 0)
      def _():
        pltpu.sync_copy(dacc.at[pl.ds(row0, rpt)],
                        deg_o.at[pl.ds(row0, rpt)])

  return agg


def _mm_split(x, w, block):
  n, din = x.shape
  dout = w.shape[1]
  dh = dout // 2

  def body(x_ref, w_ref, lo_ref, hi_ref):
    p = jnp.dot(x_ref[...], w_ref[...], preferred_element_type=jnp.float32)
    lo_ref[...] = p[:, :dh]
    hi_ref[...] = p[:, dh:]

  return pl.pallas_call(
      body,
      grid=(n // block,),
      in_specs=[pl.BlockSpec((block, din), lambda i: (i, 0)),
                pl.BlockSpec((din, dout), lambda i: (0, 0))],
      out_specs=[pl.BlockSpec((block, dh), lambda i: (i, 0)),
                 pl.BlockSpec((block, dh), lambda i: (i, 0))],
      out_shape=[jax.ShapeDtypeStruct((n, dh), jnp.float32),
                 jax.ShapeDtypeStruct((n, dh), jnp.float32)],
  )(x, w)


def _mm_bias(x, w, b, block):
  n, din = x.shape
  dout = w.shape[1]

  def body(x_ref, w_ref, b_ref, o_ref):
    o_ref[...] = jnp.dot(x_ref[...], w_ref[...],
                         preferred_element_type=jnp.float32) + b_ref[...]

  return pl.pallas_call(
      body,
      grid=(n // block,),
      in_specs=[pl.BlockSpec((block, din), lambda i: (i, 0)),
                pl.BlockSpec((din, dout), lambda i: (0, 0)),
                pl.BlockSpec((1, dout), lambda i: (0, 0))],
      out_specs=pl.BlockSpec((block, dout), lambda i: (i, 0)),
      out_shape=jax.ShapeDtypeStruct((n, dout), jnp.float32),
  )(x, w, b)


def _combine1(self1, ag1, dg, w_neigh2, block):
  n, dh = self1.shape
  d2 = w_neigh2.shape[1]
  d2h = d2 // 2

  def body(s_ref, ag_ref, dg_ref, wn_ref, h_ref, plo_ref, phi_ref):
    inv = 1.0 / jnp.maximum(dg_ref[...], 1.0)
    h = jnp.maximum(s_ref[...] + ag_ref[...] * inv, 0.0)
    h_ref[...] = h
    p2 = jnp.dot(h, wn_ref[...], preferred_element_type=jnp.float32)
    plo_ref[...] = p2[:, :d2h]
    phi_ref[...] = p2[:, d2h:]

  return pl.pallas_call(
      body,
      grid=(n // block,),
      in_specs=[pl.BlockSpec((block, dh), lambda i: (i, 0)),
                pl.BlockSpec((block, dh), lambda i: (i, 0)),
                pl.BlockSpec((block, 1), lambda i: (i, 0)),
                pl.BlockSpec((dh, d2), lambda i: (0, 0))],
      out_specs=[pl.BlockSpec((block, dh), lambda i: (i, 0)),
                 pl.BlockSpec((block, d2h), lambda i: (i, 0)),
                 pl.BlockSpec((block, d2h), lambda i: (i, 0))],
      out_shape=[jax.ShapeDtypeStruct((n, dh), jnp.float32),
                 jax.ShapeDtypeStruct((n, d2h), jnp.float32),
                 jax.ShapeDtypeStruct((n, d2h), jnp.float32)],
  )(self1, ag1, dg, w_neigh2)


def _combine2(self2, ag2, dg, block):
  n, d2 = self2.shape

  def body(s_ref, ag_ref, dg_ref, o_ref):
    inv = 1.0 / jnp.maximum(dg_ref[...], 1.0)
    o_ref[...] = s_ref[...] + ag_ref[...] * inv

  return pl.pallas_call(
      body,
      grid=(n // block,),
      in_specs=[pl.BlockSpec((block, d2), lambda i: (i, 0)),
                pl.BlockSpec((block, d2), lambda i: (i, 0)),
                pl.BlockSpec((block, 1), lambda i: (i, 0))],
      out_specs=pl.BlockSpec((block, d2), lambda i: (i, 0)),
      out_shape=jax.ShapeDtypeStruct((n, d2), jnp.float32),
  )(self2, ag2, dg)


def kernel(x, edge_index, W_self1, W_neigh1, b1, W_self2, W_neigh2, b2):
  n, _ = x.shape
  e = edge_index.shape[1]
  dh = W_neigh1.shape[1]
  dout = W_neigh2.shape[1]

  rpt = -(-(n + 1) // (NS * ZR)) * ZR
  n_acc = NS * rpt
  e_pad = -(-e // (NS * K * G)) * (NS * K * G)

  src = edge_index[0].astype(jnp.int32)
  dst = edge_index[1].astype(jnp.int32)
  pad = e_pad - e
  if pad:
    fill = jnp.arange(pad, dtype=jnp.int32)
    src = jnp.concatenate([src, (fill & 8191) if n > 8191 else (fill % n)])
    dst = jnp.concatenate([dst, n + jnp.minimum(fill & 127, n_acc - n - 1)])
  src = src.reshape(e_pad // K, K)
  dst = dst.reshape(e_pad // K, K)

  block = 2000 if n % 2000 == 0 else 8

  p1lo, p1hi = _mm_split(x, W_neigh1, block)
  ag1, dg = _make_agg(n_acc, e_pad, dh // 2, True)(p1lo, p1hi, src, dst)
  self1 = _mm_bias(x, W_self1, b1.reshape(1, dh), block)
  dg = dg.reshape(n_acc, 1)
  h1, p2lo, p2hi = _combine1(self1, ag1, dg, W_neigh2, block)

  ag2, = _make_agg(n_acc, e_pad, dout // 2, False)(p2lo, p2hi, src, dst)
  self2 = _mm_bias(h1, W_self2, b2.reshape(1, dout), block)
  return _combine2(self2, ag2, dg, block)


# speedup vs baseline: 14.6276x; 1.0798x over previous
import functools

import jax
import jax.numpy as jnp
from jax import lax
from jax.experimental import pallas as pl
from jax.experimental.pallas import tpu as pltpu
from jax.experimental.pallas import tpu_sc as plsc

NC = 2
NS = 16
NW = NC * NS
K = 128
G = 16
NB = 6
GA = 3
ZR = 64


def _make_agg(n_acc, e_pad, d2, with_deg):
  n_chunks = e_pad // (NS * K)
  rpt = n_acc // NS
  mesh = plsc.VectorSubcoreMesh(core_axis_name="c", subcore_axis_name="s",
                                num_cores=NC, num_subcores=NS)
  out_type = [jax.ShapeDtypeStruct((n_acc, 2 * d2), jnp.float32)]
  if with_deg:
    out_type += [jax.ShapeDtypeStruct((n_acc,), jnp.float32)]
  scratch = [pltpu.VMEM((2, G, K), jnp.int32),
             pltpu.VMEM((2, G, K), jnp.int32),
             pltpu.VMEM((NB, K, d2), jnp.float32),
             pltpu.VMEM((ZR, d2), jnp.float32),
             pltpu.VMEM_SHARED((n_acc, d2), jnp.float32)]
  scratch += [pltpu.SemaphoreType.DMA] * (2 * NB + 2)
  if with_deg:
    scratch += [pltpu.VMEM((K,), jnp.float32),
                pltpu.VMEM((rpt,), jnp.float32),
                pltpu.VMEM_SHARED((n_acc,), jnp.float32)]

  @functools.partial(pl.kernel, mesh=mesh, out_type=out_type,
                     scratch_types=scratch,
                     compiler_params=pltpu.CompilerParams(
                         use_tc_tiling_on_sc=False))
  def agg(p_lo, p_hi, src_hbm, dst_hbm, *refs):
    nsem = 2 * NB + 2
    if with_deg:
      (out, deg_o, src_idx, dst_idx, rows, zblk, acc) = refs[:7]
      sems = refs[7:7 + nsem]
      ones, dzero, dacc = refs[7 + nsem:]
    else:
      (out, src_idx, dst_idx, rows, zblk, acc) = refs[:6]
      sems = refs[6:6 + nsem]
    gsems, ssems = sems[:NB], sems[NB:2 * NB]
    dsem, isem = sems[2 * NB], sems[2 * NB + 1]
    c = lax.axis_index("c")
    s = lax.axis_index("s")
    row0 = s * rpt

    def zrow(i, _):
      for kk in range(d2 // 16):
        zblk[i, pl.ds(kk * 16, 16)] = jnp.zeros((16,), jnp.float32)
      return 0
    lax.fori_loop(0, ZR, zrow, 0)

    def zacc(i, _):
      pltpu.sync_copy(zblk, acc.at[pl.ds(row0 + i * ZR, ZR)])
      return 0
    lax.fori_loop(0, rpt // ZR, zacc, 0)

    if with_deg:
      for kk in range(K // 16):
        ones[pl.ds(kk * 16, 16)] = jnp.ones((16,), jnp.float32)

      def zdeg(i, _):
        dzero[pl.ds(i * 16, 16)] = jnp.zeros((16,), jnp.float32)
        return 0
      lax.fori_loop(0, rpt // 16, zdeg, 0)
      pltpu.sync_copy(dzero, dacc.at[pl.ds(row0, rpt)])
    plsc.subcore_barrier()

    crow = s * n_chunks
    ngroups = n_chunks // G
    pltpu.sync_copy(src_hbm.at[pl.ds(crow, G)], src_idx.at[0])
    pltpu.sync_copy(dst_hbm.at[pl.ds(crow, G)], dst_idx.at[0])

    def group(g, _):
      gb = g % 2
      rn = crow + jnp.minimum(g + 1, ngroups - 1) * G
      di1 = pltpu.async_copy(src_hbm.at[pl.ds(rn, G)], src_idx.at[1 - gb],
                             isem)
      di2 = pltpu.async_copy(dst_hbm.at[pl.ds(rn, G)], dst_idx.at[1 - gb],
                             isem)
      src_g = src_idx.at[gb]
      dst_g = dst_idx.at[gb]
      sd, dd = {}, {}
      waited = set()
      state = {"nf": 0}

      def fire_until(limit):
        while state["nf"] < G and state["nf"] < limit:
          j = state["nf"]
          prev = j - NB
          if prev >= 0 and prev not in waited:
            sd[prev].wait()
            waited.add(prev)
          idx_r = src_g.at[j]
          buf_r = rows.at[j % NB]
          sem = gsems[j % NB]

          @pl.when(c == 0)
          def _():
            pltpu.async_copy(p_lo.at[idx_r], buf_r, sem)

          @pl.when(c == 1)
          def _():
            pltpu.async_copy(p_hi.at[idx_r], buf_r, sem)
          state["nf"] = j + 1

      fire_until(GA)
      for k in range(G):
        fire_until(k + 1 + GA)
        pltpu.make_async_copy(p_lo.at[src_g.at[k]], rows.at[k % NB],
                              gsems[k % NB]).wait()
        sd[k] = pltpu.async_copy(rows.at[k % NB], acc.at[dst_g.at[k]],
                                 ssems[k % NB], add=True)
        if with_deg:
          dd[k] = pltpu.async_copy(ones, dacc.at[dst_g.at[k]], dsem,
                                   add=True)
      for k in range(G):
        if k not in waited:
          sd[k].wait()
        if with_deg:
          dd[k].wait()
      di1.wait()
      di2.wait()
      return 0
    lax.fori_loop(0, n_chunks // G, group, 0)
    plsc.subcore_barrier()

    d2c = c * d2
    pltpu.sync_copy(acc.at[pl.ds(row0, rpt)],
                    out.at[pl.ds(row0, rpt), pl.ds(d2c, d2)])
    if with_deg:
      @pl.when(c == 0)
      def _():
        pltpu.sync_copy(dacc.at[pl.ds(row0, rpt)],
                        deg_o.at[pl.ds(row0, rpt)])

  return agg


def _proj1(x, w_neigh, w_self, b, block):
  n, din = x.shape
  dout = w_neigh.shape[1]
  dh = dout // 2
  ds_out = w_self.shape[1]

  def body(x_ref, wn_ref, ws_ref, b_ref, lo_ref, hi_ref, s_ref):
    xv = x_ref[...]
    p = jnp.dot(xv, wn_ref[...], preferred_element_type=jnp.float32)
    lo_ref[...] = p[:, :dh]
    hi_ref[...] = p[:, dh:]
    s_ref[...] = jnp.dot(xv, ws_ref[...],
                         preferred_element_type=jnp.float32) + b_ref[...]

  return pl.pallas_call(
      body,
      grid=(n // block,),
      in_specs=[pl.BlockSpec((block, din), lambda i: (i, 0)),
                pl.BlockSpec((din, dout), lambda i: (0, 0)),
                pl.BlockSpec((din, ds_out), lambda i: (0, 0)),
                pl.BlockSpec((1, ds_out), lambda i: (0, 0))],
      out_specs=[pl.BlockSpec((block, dh), lambda i: (i, 0)),
                 pl.BlockSpec((block, dh), lambda i: (i, 0)),
                 pl.BlockSpec((block, ds_out), lambda i: (i, 0))],
      out_shape=[jax.ShapeDtypeStruct((n, dh), jnp.float32),
                 jax.ShapeDtypeStruct((n, dh), jnp.float32),
                 jax.ShapeDtypeStruct((n, ds_out), jnp.float32)],
  )(x, w_neigh, w_self, b)


def _combine1(self1, ag1, dg, w_neigh2, w_self2, b2, block):
  n, dh = self1.shape
  d2 = w_neigh2.shape[1]
  d2h = d2 // 2

  def body(s_ref, ag_ref, dg_ref, wn_ref, ws_ref, b_ref,
           plo_ref, phi_ref, s2_ref):
    inv = 1.0 / jnp.maximum(dg_ref[...], 1.0)
    h = jnp.maximum(s_ref[...] + ag_ref[...] * inv, 0.0)
    p2 = jnp.dot(h, wn_ref[...], preferred_element_type=jnp.float32)
    plo_ref[...] = p2[:, :d2h]
    phi_ref[...] = p2[:, d2h:]
    s2_ref[...] = jnp.dot(h, ws_ref[...],
                          preferred_element_type=jnp.float32) + b_ref[...]

  return pl.pallas_call(
      body,
      grid=(n // block,),
      in_specs=[pl.BlockSpec((block, dh), lambda i: (i, 0)),
                pl.BlockSpec((block, dh), lambda i: (i, 0)),
                pl.BlockSpec((block, 1), lambda i: (i, 0)),
                pl.BlockSpec((dh, d2), lambda i: (0, 0)),
                pl.BlockSpec((dh, d2), lambda i: (0, 0)),
                pl.BlockSpec((1, d2), lambda i: (0, 0))],
      out_specs=[pl.BlockSpec((block, d2h), lambda i: (i, 0)),
                 pl.BlockSpec((block, d2h), lambda i: (i, 0)),
                 pl.BlockSpec((block, d2), lambda i: (i, 0))],
      out_shape=[jax.ShapeDtypeStruct((n, d2h), jnp.float32),
                 jax.ShapeDtypeStruct((n, d2h), jnp.float32),
                 jax.ShapeDtypeStruct((n, d2), jnp.float32)],
  )(self1, ag1, dg, w_neigh2, w_self2, b2)


def _combine2(self2, ag2, dg, block):
  n, d2 = self2.shape

  def body(s_ref, ag_ref, dg_ref, o_ref):
    inv = 1.0 / jnp.maximum(dg_ref[...], 1.0)
    o_ref[...] = s_ref[...] + ag_ref[...] * inv

  return pl.pallas_call(
      body,
      grid=(n // block,),
      in_specs=[pl.BlockSpec((block, d2), lambda i: (i, 0)),
                pl.BlockSpec((block, d2), lambda i: (i, 0)),
                pl.BlockSpec((block, 1), lambda i: (i, 0))],
      out_specs=pl.BlockSpec((block, d2), lambda i: (i, 0)),
      out_shape=jax.ShapeDtypeStruct((n, d2), jnp.float32),
  )(self2, ag2, dg)


def kernel(x, edge_index, W_self1, W_neigh1, b1, W_self2, W_neigh2, b2):
  n, _ = x.shape
  e = edge_index.shape[1]
  dh = W_neigh1.shape[1]
  dout = W_neigh2.shape[1]

  rpt = -(-(n + 1) // (NS * ZR)) * ZR
  n_acc = NS * rpt
  e_pad = -(-e // (NS * K * G)) * (NS * K * G)

  src = edge_index[0].astype(jnp.int32)
  dst = edge_index[1].astype(jnp.int32)
  pad = e_pad - e
  if pad:
    fill = jnp.arange(pad, dtype=jnp.int32)
    src = jnp.concatenate([src, (fill & 8191) if n > 8191 else (fill % n)])
    dst = jnp.concatenate([dst, n + jnp.minimum(fill & 127, n_acc - n - 1)])
  src = src.reshape(e_pad // K, K)
  dst = dst.reshape(e_pad // K, K)

  block = 2000 if n % 2000 == 0 else 8

  p1lo, p1hi, self1 = _proj1(x, W_neigh1, W_self1, b1.reshape(1, dh), block)
  ag1, dg = _make_agg(n_acc, e_pad, dh // 2, True)(p1lo, p1hi, src, dst)
  dg = dg.reshape(n_acc, 1)
  p2lo, p2hi, self2 = _combine1(self1, ag1, dg, W_neigh2, W_self2,
                                b2.reshape(1, dout), block)

  ag2, = _make_agg(n_acc, e_pad, dout // 2, False)(p2lo, p2hi, src, dst)
  return _combine2(self2, ag2, dg, block)


# speedup vs baseline: 15.1081x; 1.0329x over previous
import functools

import jax
import jax.numpy as jnp
from jax import lax
from jax.experimental import pallas as pl
from jax.experimental.pallas import tpu as pltpu
from jax.experimental.pallas import tpu_sc as plsc

NC = 2
NS = 16
NW = NC * NS
K = 128
G = 32
NB = 6
GA = 3
ZR = 64


def _make_agg(n_acc, e_pad, d2, with_deg):
  n_chunks = e_pad // (NS * K)
  rpt = n_acc // NS
  mesh = plsc.VectorSubcoreMesh(core_axis_name="c", subcore_axis_name="s",
                                num_cores=NC, num_subcores=NS)
  out_type = [jax.ShapeDtypeStruct((n_acc, 2 * d2), jnp.float32)]
  if with_deg:
    out_type += [jax.ShapeDtypeStruct((n_acc,), jnp.float32)]
  scratch = [pltpu.VMEM((2, G, K), jnp.int32),
             pltpu.VMEM((2, G, K), jnp.int32),
             pltpu.VMEM((NB, K, d2), jnp.float32),
             pltpu.VMEM((ZR, d2), jnp.float32),
             pltpu.VMEM_SHARED((n_acc, d2), jnp.float32)]
  scratch += [pltpu.SemaphoreType.DMA] * (2 * NB + 2)
  if with_deg:
    scratch += [pltpu.VMEM((K,), jnp.float32),
                pltpu.VMEM((rpt,), jnp.float32),
                pltpu.VMEM_SHARED((n_acc,), jnp.float32)]

  @functools.partial(pl.kernel, mesh=mesh, out_type=out_type,
                     scratch_types=scratch,
                     compiler_params=pltpu.CompilerParams(
                         use_tc_tiling_on_sc=False))
  def agg(p_lo, p_hi, src_hbm, dst_hbm, *refs):
    nsem = 2 * NB + 2
    if with_deg:
      (out, deg_o, src_idx, dst_idx, rows, zblk, acc) = refs[:7]
      sems = refs[7:7 + nsem]
      ones, dzero, dacc = refs[7 + nsem:]
    else:
      (out, src_idx, dst_idx, rows, zblk, acc) = refs[:6]
      sems = refs[6:6 + nsem]
    gsems, ssems = sems[:NB], sems[NB:2 * NB]
    dsem, isem = sems[2 * NB], sems[2 * NB + 1]
    c = lax.axis_index("c")
    s = lax.axis_index("s")
    row0 = s * rpt

    def zrow(i, _):
      for kk in range(d2 // 16):
        zblk[i, pl.ds(kk * 16, 16)] = jnp.zeros((16,), jnp.float32)
      return 0
    lax.fori_loop(0, ZR, zrow, 0)

    def zacc(i, _):
      pltpu.sync_copy(zblk, acc.at[pl.ds(row0 + i * ZR, ZR)])
      return 0
    lax.fori_loop(0, rpt // ZR, zacc, 0)

    if with_deg:
      for kk in range(K // 16):
        ones[pl.ds(kk * 16, 16)] = jnp.ones((16,), jnp.float32)

      def zdeg(i, _):
        dzero[pl.ds(i * 16, 16)] = jnp.zeros((16,), jnp.float32)
        return 0
      lax.fori_loop(0, rpt // 16, zdeg, 0)
      pltpu.sync_copy(dzero, dacc.at[pl.ds(row0, rpt)])
    plsc.subcore_barrier()

    crow = s * n_chunks
    ngroups = n_chunks // G
    pltpu.sync_copy(src_hbm.at[pl.ds(crow, G)], src_idx.at[0])
    pltpu.sync_copy(dst_hbm.at[pl.ds(crow, G)], dst_idx.at[0])

    def group(g, _):
      gb = g % 2
      rn = crow + jnp.minimum(g + 1, ngroups - 1) * G
      di1 = pltpu.async_copy(src_hbm.at[pl.ds(rn, G)], src_idx.at[1 - gb],
                             isem)
      di2 = pltpu.async_copy(dst_hbm.at[pl.ds(rn, G)], dst_idx.at[1 - gb],
                             isem)
      src_g = src_idx.at[gb]
      dst_g = dst_idx.at[gb]
      sd, dd = {}, {}
      waited = set()
      state = {"nf": 0}

      def fire_until(limit):
        while state["nf"] < G and state["nf"] < limit:
          j = state["nf"]
          prev = j - NB
          if prev >= 0 and prev not in waited:
            sd[prev].wait()
            waited.add(prev)
          idx_r = src_g.at[j]
          buf_r = rows.at[j % NB]
          sem = gsems[j % NB]

          @pl.when(c == 0)
          def _():
            pltpu.async_copy(p_lo.at[idx_r], buf_r, sem)

          @pl.when(c == 1)
          def _():
            pltpu.async_copy(p_hi.at[idx_r], buf_r, sem)
          state["nf"] = j + 1

      fire_until(GA)
      for k in range(G):
        fire_until(k + 1 + GA)
        pltpu.make_async_copy(p_lo.at[src_g.at[k]], rows.at[k % NB],
                              gsems[k % NB]).wait()
        sd[k] = pltpu.async_copy(rows.at[k % NB], acc.at[dst_g.at[k]],
                                 ssems[k % NB], add=True)
        if with_deg:
          dd[k] = pltpu.async_copy(ones, dacc.at[dst_g.at[k]], dsem,
                                   add=True)
      for k in range(G):
        if k not in waited:
          sd[k].wait()
        if with_deg:
          dd[k].wait()
      di1.wait()
      di2.wait()
      return 0
    lax.fori_loop(0, n_chunks // G, group, 0)
    plsc.subcore_barrier()

    d2c = c * d2
    pltpu.sync_copy(acc.at[pl.ds(row0, rpt)],
                    out.at[pl.ds(row0, rpt), pl.ds(d2c, d2)])
    if with_deg:
      @pl.when(c == 0)
      def _():
        pltpu.sync_copy(dacc.at[pl.ds(row0, rpt)],
                        deg_o.at[pl.ds(row0, rpt)])

  return agg


def _proj1(x, w_neigh, w_self, b, block):
  n, din = x.shape
  dout = w_neigh.shape[1]
  dh = dout // 2
  ds_out = w_self.shape[1]

  def body(x_ref, wn_ref, ws_ref, b_ref, lo_ref, hi_ref, s_ref):
    xv = x_ref[...]
    p = jnp.dot(xv, wn_ref[...], preferred_element_type=jnp.float32)
    lo_ref[...] = p[:, :dh]
    hi_ref[...] = p[:, dh:]
    s_ref[...] = jnp.dot(xv, ws_ref[...],
                         preferred_element_type=jnp.float32) + b_ref[...]

  return pl.pallas_call(
      body,
      grid=(n // block,),
      in_specs=[pl.BlockSpec((block, din), lambda i: (i, 0)),
                pl.BlockSpec((din, dout), lambda i: (0, 0)),
                pl.BlockSpec((din, ds_out), lambda i: (0, 0)),
                pl.BlockSpec((1, ds_out), lambda i: (0, 0))],
      out_specs=[pl.BlockSpec((block, dh), lambda i: (i, 0)),
                 pl.BlockSpec((block, dh), lambda i: (i, 0)),
                 pl.BlockSpec((block, ds_out), lambda i: (i, 0))],
      out_shape=[jax.ShapeDtypeStruct((n, dh), jnp.float32),
                 jax.ShapeDtypeStruct((n, dh), jnp.float32),
                 jax.ShapeDtypeStruct((n, ds_out), jnp.float32)],
  )(x, w_neigh, w_self, b)


def _combine1(self1, ag1, dg, w_neigh2, w_self2, b2, block):
  n, dh = self1.shape
  d2 = w_neigh2.shape[1]
  d2h = d2 // 2

  def body(s_ref, ag_ref, dg_ref, wn_ref, ws_ref, b_ref,
           plo_ref, phi_ref, s2_ref):
    inv = 1.0 / jnp.maximum(dg_ref[...], 1.0)
    h = jnp.maximum(s_ref[...] + ag_ref[...] * inv, 0.0)
    p2 = jnp.dot(h, wn_ref[...], preferred_element_type=jnp.float32)
    plo_ref[...] = p2[:, :d2h]
    phi_ref[...] = p2[:, d2h:]
    s2_ref[...] = jnp.dot(h, ws_ref[...],
                          preferred_element_type=jnp.float32) + b_ref[...]

  return pl.pallas_call(
      body,
      grid=(n // block,),
      in_specs=[pl.BlockSpec((block, dh), lambda i: (i, 0)),
                pl.BlockSpec((block, dh), lambda i: (i, 0)),
                pl.BlockSpec((block, 1), lambda i: (i, 0)),
                pl.BlockSpec((dh, d2), lambda i: (0, 0)),
                pl.BlockSpec((dh, d2), lambda i: (0, 0)),
                pl.BlockSpec((1, d2), lambda i: (0, 0))],
      out_specs=[pl.BlockSpec((block, d2h), lambda i: (i, 0)),
                 pl.BlockSpec((block, d2h), lambda i: (i, 0)),
                 pl.BlockSpec((block, d2), lambda i: (i, 0))],
      out_shape=[jax.ShapeDtypeStruct((n, d2h), jnp.float32),
                 jax.ShapeDtypeStruct((n, d2h), jnp.float32),
                 jax.ShapeDtypeStruct((n, d2), jnp.float32)],
  )(self1, ag1, dg, w_neigh2, w_self2, b2)


def _combine2(self2, ag2, dg, block):
  n, d2 = self2.shape

  def body(s_ref, ag_ref, dg_ref, o_ref):
    inv = 1.0 / jnp.maximum(dg_ref[...], 1.0)
    o_ref[...] = s_ref[...] + ag_ref[...] * inv

  return pl.pallas_call(
      body,
      grid=(n // block,),
      in_specs=[pl.BlockSpec((block, d2), lambda i: (i, 0)),
                pl.BlockSpec((block, d2), lambda i: (i, 0)),
                pl.BlockSpec((block, 1), lambda i: (i, 0))],
      out_specs=pl.BlockSpec((block, d2), lambda i: (i, 0)),
      out_shape=jax.ShapeDtypeStruct((n, d2), jnp.float32),
  )(self2, ag2, dg)


def kernel(x, edge_index, W_self1, W_neigh1, b1, W_self2, W_neigh2, b2):
  n, _ = x.shape
  e = edge_index.shape[1]
  dh = W_neigh1.shape[1]
  dout = W_neigh2.shape[1]

  rpt = -(-(n + 1) // (NS * ZR)) * ZR
  n_acc = NS * rpt
  e_pad = -(-e // (NS * K * G)) * (NS * K * G)

  src = edge_index[0].astype(jnp.int32)
  dst = edge_index[1].astype(jnp.int32)
  pad = e_pad - e
  if pad:
    fill = jnp.arange(pad, dtype=jnp.int32)
    src = jnp.concatenate([src, (fill & 8191) if n > 8191 else (fill % n)])
    dst = jnp.concatenate([dst, n + jnp.minimum(fill & 127, n_acc - n - 1)])
  src = src.reshape(e_pad // K, K)
  dst = dst.reshape(e_pad // K, K)

  block = 2000 if n % 2000 == 0 else 8

  p1lo, p1hi, self1 = _proj1(x, W_neigh1, W_self1, b1.reshape(1, dh), block)
  ag1, dg = _make_agg(n_acc, e_pad, dh // 2, True)(p1lo, p1hi, src, dst)
  dg = dg.reshape(n_acc, 1)
  p2lo, p2hi, self2 = _combine1(self1, ag1, dg, W_neigh2, W_self2,
                                b2.reshape(1, dout), block)

  ag2, = _make_agg(n_acc, e_pad, dout // 2, False)(p2lo, p2hi, src, dst)
  return _combine2(self2, ag2, dg, block)
